# Initial kernel scaffold; baseline (speedup 1.0000x reference)
import jax
import jax.numpy as jnp
from jax.experimental import pallas as pl


def kernel(x, edge_index, W1, b1, g1, be1, W2, b2, g2, be2):
    raise NotImplementedError("write your pallas kernel here")


# speedup vs baseline: 4.6591x; 4.6591x over previous
import functools

import jax
import jax.numpy as jnp
from jax import lax
from jax.experimental import pallas as pl
from jax.experimental.pallas import tpu as pltpu
from jax.experimental.pallas import tpu_sc as plsc

_NC, _NS = 2, 16
_NW = _NC * _NS
_CH = 80
_WCH = 128
_RPT = 640
_BN = 1000


def _scatter_partials(h, row, col, zslab):
    N, D = h.shape
    E = row.shape[0]
    ept = E // _NW
    nchunk = ept // _CH
    rpt = _RPT
    npad = _NS * _RPT
    nwch = rpt // _WCH

    mesh = plsc.VectorSubcoreMesh(core_axis_name="c", subcore_axis_name="s")

    @functools.partial(
        pl.kernel,
        out_type=jax.ShapeDtypeStruct((_NC, npad, D), jnp.float32),
        mesh=mesh,
        scratch_types=[
            pltpu.VMEM((_WCH, D), jnp.float32),
            pltpu.VMEM((_CH,), jnp.int32),
            pltpu.VMEM((_CH,), jnp.int32),
            pltpu.VMEM((_CH, D), jnp.float32),
            pltpu.VMEM_SHARED((npad, D), jnp.float32),
            pltpu.SemaphoreType.DMA,
        ],
    )
    def k(h_hbm, row_hbm, col_hbm, z_hbm, out_hbm, slab, colv, rowv, rows, acc, sem):
        c = lax.axis_index("c")
        s = lax.axis_index("s")
        wid = s * _NC + c
        rbase = s * rpt

        pltpu.sync_copy(z_hbm, slab)

        def zbody(j, carry):
            pltpu.sync_copy(slab, acc.at[pl.ds(rbase + j * _WCH, _WCH)])
            return carry

        lax.fori_loop(0, nwch, zbody, 0)
        plsc.subcore_barrier()

        ebase = wid * ept

        def body(i, carry):
            off = ebase + i * _CH
            pltpu.sync_copy(col_hbm.at[pl.ds(off, _CH)], colv)
            pltpu.sync_copy(row_hbm.at[pl.ds(off, _CH)], rowv)
            pltpu.async_copy(h_hbm.at[colv], rows, sem).wait()
            pltpu.sync_copy(rows, acc.at[rowv], add=True)
            return carry

        lax.fori_loop(0, nchunk, body, 0)
        plsc.subcore_barrier()

        def wbody(j, carry):
            r0 = rbase + j * _WCH
            pltpu.sync_copy(acc.at[pl.ds(r0, _WCH)], slab)
            pltpu.sync_copy(slab, out_hbm.at[c, pl.ds(r0, _WCH)])
            return carry

        lax.fori_loop(0, nwch, wbody, 0)

    return k(h, row, col, zslab)


def _matmul(x, W):
    N, D = x.shape
    H = W.shape[1]

    def kfn(x_ref, w_ref, o_ref):
        o_ref[...] = jnp.dot(x_ref[...], w_ref[...],
                             preferred_element_type=jnp.float32)

    return pl.pallas_call(
        kfn,
        grid=(N // _BN,),
        in_specs=[
            pl.BlockSpec((_BN, D), lambda i: (i, 0)),
            pl.BlockSpec((D, H), lambda i: (0, 0)),
        ],
        out_specs=pl.BlockSpec((_BN, H), lambda i: (i, 0)),
        out_shape=jax.ShapeDtypeStruct((N, H), jnp.float32),
    )(x, W)


def _mid(p, b1, g1, be1, W2, N):
    H = p.shape[2]
    D2 = W2.shape[1]

    def kfn(p_ref, b_ref, g_ref, be_ref, w_ref, o_ref):
        s = p_ref[0] + p_ref[1] + b_ref[...]
        mu = jnp.mean(s, axis=-1, keepdims=True)
        var = jnp.mean((s - mu) ** 2, axis=-1, keepdims=True)
        t = (s - mu) * lax.rsqrt(var + 1e-5) * g_ref[...] + be_ref[...]
        t = jnp.maximum(t, 0.0)
        o_ref[...] = jnp.dot(t, w_ref[...], preferred_element_type=jnp.float32)

    vec = lambda i: (0, 0)
    return pl.pallas_call(
        kfn,
        grid=(N // _BN,),
        in_specs=[
            pl.BlockSpec((2, _BN, H), lambda i: (0, i, 0)),
            pl.BlockSpec((1, H), vec),
            pl.BlockSpec((1, H), vec),
            pl.BlockSpec((1, H), vec),
            pl.BlockSpec((H, D2), vec),
        ],
        out_specs=pl.BlockSpec((_BN, D2), lambda i: (i, 0)),
        out_shape=jax.ShapeDtypeStruct((N, D2), jnp.float32),
    )(p, b1.reshape(1, H), g1.reshape(1, H), be1.reshape(1, H), W2)


def _final(p, b2, g2, be2, x):
    N, D = x.shape

    def kfn(p_ref, b_ref, g_ref, be_ref, x_ref, o_ref):
        s = p_ref[0] + p_ref[1] + b_ref[...]
        mu = jnp.mean(s, axis=-1, keepdims=True)
        var = jnp.mean((s - mu) ** 2, axis=-1, keepdims=True)
        t = (s - mu) * lax.rsqrt(var + 1e-5) * g_ref[...] + be_ref[...]
        o_ref[...] = t + x_ref[...]

    vec = lambda i: (0, 0)
    return pl.pallas_call(
        kfn,
        grid=(N // _BN,),
        in_specs=[
            pl.BlockSpec((2, _BN, D), lambda i: (0, i, 0)),
            pl.BlockSpec((1, D), vec),
            pl.BlockSpec((1, D), vec),
            pl.BlockSpec((1, D), vec),
            pl.BlockSpec((_BN, D), lambda i: (i, 0)),
        ],
        out_specs=pl.BlockSpec((_BN, D), lambda i: (i, 0)),
        out_shape=jax.ShapeDtypeStruct((N, D), jnp.float32),
    )(p, b2.reshape(1, D), g2.reshape(1, D), be2.reshape(1, D), x)


def kernel(x, edge_index, W1, b1, g1, be1, W2, b2, g2, be2):
    N, D = x.shape
    H = W1.shape[1]
    row = edge_index[0]
    col = edge_index[1]
    zslab = jnp.zeros((_WCH, H), jnp.float32)

    h1 = _matmul(x, W1)
    p1 = _scatter_partials(h1, row, col, zslab)
    h2 = _mid(p1, b1, g1, be1, W2, N)
    p2 = _scatter_partials(h2, row, col, zslab)
    return _final(p2, b2, g2, be2, x)


# speedup vs baseline: 5.4646x; 1.1729x over previous
import functools

import jax
import jax.numpy as jnp
from jax import lax
from jax.experimental import pallas as pl
from jax.experimental.pallas import tpu as pltpu
from jax.experimental.pallas import tpu_sc as plsc

_NC, _NS = 2, 16
_NW = _NC * _NS
_CH = 128
_NB = 4
_WCH = 128
_RPT = 640
_BN = 1000


def _scatter_halves(h2, row3, col3, zslab):
    Dh = h2.shape[2]
    nchunk = row3.shape[1]
    rpt = _RPT
    npad = _NS * _RPT
    nwch = rpt // _WCH

    mesh = plsc.VectorSubcoreMesh(core_axis_name="c", subcore_axis_name="s")

    @functools.partial(
        pl.kernel,
        out_type=jax.ShapeDtypeStruct((_NC, npad, Dh), jnp.float32),
        mesh=mesh,
        scratch_types=[
            pltpu.VMEM((_WCH, Dh), jnp.float32),
            pltpu.VMEM((nchunk, _CH), jnp.int32),
            pltpu.VMEM((nchunk, _CH), jnp.int32),
            [pltpu.VMEM((_CH, Dh), jnp.float32) for _ in range(_NB)],
            pltpu.VMEM_SHARED((npad, Dh), jnp.float32),
            [pltpu.SemaphoreType.DMA for _ in range(_NB)],
        ],
        compiler_params=pltpu.CompilerParams(use_tc_tiling_on_sc=False),
    )
    def k(h_hbm, row_hbm, col_hbm, z_hbm, out_hbm, slab, cols, rows_i, bufs,
          acc, gsem):
        c = lax.axis_index("c")
        s = lax.axis_index("s")
        rbase = s * rpt
        hsrc = h_hbm.at[c]

        pltpu.sync_copy(col_hbm.at[s], cols)
        pltpu.sync_copy(row_hbm.at[s], rows_i)

        pltpu.sync_copy(z_hbm, slab)

        def zbody(j, carry):
            pltpu.sync_copy(slab, acc.at[pl.ds(rbase + j * _WCH, _WCH)])
            return carry

        lax.fori_loop(0, nwch, zbody, 0)
        plsc.subcore_barrier()

        for b in range(_NB):
            pltpu.async_copy(hsrc.at[cols.at[b]], bufs[b], gsem[b])

        def body(go, carry):
            for b in range(_NB):
                g = go * _NB + b
                pltpu.make_async_copy(hsrc.at[cols.at[g]], bufs[b],
                                      gsem[b]).wait()
                pltpu.sync_copy(bufs[b], acc.at[rows_i.at[g]], add=True)

                @pl.when(go * _NB + b + _NB < nchunk)
                def _():
                    pltpu.async_copy(hsrc.at[cols.at[g + _NB]], bufs[b],
                                     gsem[b])
            return carry

        lax.fori_loop(0, nchunk // _NB, body, 0)
        plsc.subcore_barrier()

        def wbody(j, carry):
            r0 = rbase + j * _WCH
            pltpu.sync_copy(acc.at[pl.ds(r0, _WCH)], slab)
            pltpu.sync_copy(slab, out_hbm.at[c, pl.ds(r0, _WCH)])
            return carry

        lax.fori_loop(0, nwch, wbody, 0)

    return k(h2, row3, col3, zslab)


def _matmul(x, W):
    N, D = x.shape
    H = W.shape[1]
    Hh = H // 2

    def kfn(x_ref, w_ref, o_ref):
        r = jnp.dot(x_ref[...], w_ref[...], preferred_element_type=jnp.float32)
        o_ref[0] = r[:, :Hh]
        o_ref[1] = r[:, Hh:]

    return pl.pallas_call(
        kfn,
        grid=(N // _BN,),
        in_specs=[
            pl.BlockSpec((_BN, D), lambda i: (i, 0)),
            pl.BlockSpec((D, H), lambda i: (0, 0)),
        ],
        out_specs=pl.BlockSpec((2, _BN, Hh), lambda i: (0, i, 0)),
        out_shape=jax.ShapeDtypeStruct((2, N, Hh), jnp.float32),
    )(x, W)


def _mid(p, b1, g1, be1, W2, N):
    Hh = p.shape[2]
    H = 2 * Hh
    D2 = W2.shape[1]
    D2h = D2 // 2

    def kfn(p_ref, b_ref, g_ref, be_ref, w_ref, o_ref):
        s = jnp.concatenate([p_ref[0], p_ref[1]], axis=-1) + b_ref[...]
        mu = jnp.mean(s, axis=-1, keepdims=True)
        var = jnp.mean((s - mu) ** 2, axis=-1, keepdims=True)
        t = (s - mu) * lax.rsqrt(var + 1e-5) * g_ref[...] + be_ref[...]
        t = jnp.maximum(t, 0.0)
        r = jnp.dot(t, w_ref[...], preferred_element_type=jnp.float32)
        o_ref[0] = r[:, :D2h]
        o_ref[1] = r[:, D2h:]

    vec = lambda i: (0, 0)
    return pl.pallas_call(
        kfn,
        grid=(N // _BN,),
        in_specs=[
            pl.BlockSpec((2, _BN, Hh), lambda i: (0, i, 0)),
            pl.BlockSpec((1, H), vec),
            pl.BlockSpec((1, H), vec),
            pl.BlockSpec((1, H), vec),
            pl.BlockSpec((H, D2), vec),
        ],
        out_specs=pl.BlockSpec((2, _BN, D2h), lambda i: (0, i, 0)),
        out_shape=jax.ShapeDtypeStruct((2, N, D2h), jnp.float32),
    )(p, b1.reshape(1, H), g1.reshape(1, H), be1.reshape(1, H), W2)


def _final(p, b2, g2, be2, x):
    N, D = x.shape
    Dh = p.shape[2]

    def kfn(p_ref, b_ref, g_ref, be_ref, x_ref, o_ref):
        s = jnp.concatenate([p_ref[0], p_ref[1]], axis=-1) + b_ref[...]
        mu = jnp.mean(s, axis=-1, keepdims=True)
        var = jnp.mean((s - mu) ** 2, axis=-1, keepdims=True)
        t = (s - mu) * lax.rsqrt(var + 1e-5) * g_ref[...] + be_ref[...]
        o_ref[...] = t + x_ref[...]

    vec = lambda i: (0, 0)
    return pl.pallas_call(
        kfn,
        grid=(N // _BN,),
        in_specs=[
            pl.BlockSpec((2, _BN, Dh), lambda i: (0, i, 0)),
            pl.BlockSpec((1, D), vec),
            pl.BlockSpec((1, D), vec),
            pl.BlockSpec((1, D), vec),
            pl.BlockSpec((_BN, D), lambda i: (i, 0)),
        ],
        out_specs=pl.BlockSpec((_BN, D), lambda i: (i, 0)),
        out_shape=jax.ShapeDtypeStruct((N, D), jnp.float32),
    )(p, b2.reshape(1, D), g2.reshape(1, D), be2.reshape(1, D), x)


def kernel(x, edge_index, W1, b1, g1, be1, W2, b2, g2, be2):
    N, D = x.shape
    H = W1.shape[1]
    E = edge_index.shape[1]
    zslab = jnp.zeros((_WCH, H // 2), jnp.float32)

    ept = E // _NS
    eptp = -(-ept // (_NB * _CH)) * (_NB * _CH)
    npad = _NS * _RPT
    row3 = jnp.full((_NS, eptp), npad - 1, jnp.int32)
    row3 = row3.at[:, :ept].set(edge_index[0].reshape(_NS, ept))
    row3 = row3.reshape(_NS, eptp // _CH, _CH)
    col3 = jnp.zeros((_NS, eptp), jnp.int32)
    col3 = col3.at[:, :ept].set(edge_index[1].reshape(_NS, ept))
    col3 = col3.reshape(_NS, eptp // _CH, _CH)

    h1 = _matmul(x, W1)
    p1 = _scatter_halves(h1, row3, col3, zslab)
    h2 = _mid(p1, b1, g1, be1, W2, N)
    p2 = _scatter_halves(h2, row3, col3, zslab)
    return _final(p2, b2, g2, be2, x)
